# baseline (speedup 1.0000x reference)
import jax
import jax.numpy as jnp
from jax import lax
from jax.experimental import pallas as pl
from jax.experimental.pallas import tpu as pltpu
from jax.experimental.pallas import tpu_sc as plsc

SEQ_LEN = 200
EMB_DIM = 128
LANES = 16
NUM_TILES = 16
ROWS_PER_TILE = 256 // NUM_TILES
CHUNKS = EMB_DIM // LANES

_mesh = plsc.VectorSubcoreMesh(
    core_axis_name="c", subcore_axis_name="s", num_cores=1
)


def _sc_body(idx_hbm, table_hbm, f_hbm, out_hbm,
             idx_v, rows_v, f_v, part_v, shared, res_v, sem):
    sid = lax.axis_index("s")
    base = sid * ROWS_PER_TILE

    pltpu.sync_copy(idx_hbm.at[pl.ds(base, ROWS_PER_TILE)], idx_v)
    pltpu.sync_copy(f_hbm.at[pl.ds(base, ROWS_PER_TILE)], f_v)
    pltpu.async_copy(table_hbm.at[idx_v], rows_v, sem).wait()

    for c in range(CHUNKS):
        col = pl.ds(c * LANES, LANES)
        acc = rows_v[0, col] * f_v[0, col]
        for r in range(1, ROWS_PER_TILE):
            acc = acc + rows_v[r, col] * f_v[r, col]
        part_v[col] = acc

    pltpu.sync_copy(part_v, shared.at[sid])
    plsc.subcore_barrier()

    @pl.when(sid == 0)
    def _():
        pltpu.sync_copy(shared, rows_v)
        for c in range(CHUNKS):
            col = pl.ds(c * LANES, LANES)
            s = rows_v[0, col]
            for t in range(1, NUM_TILES):
                s = s + rows_v[t, col]
            res_v[0, col] = s
        pltpu.sync_copy(res_v, out_hbm)


_sc_call = pl.kernel(
    _sc_body,
    out_type=jax.ShapeDtypeStruct((1, EMB_DIM), jnp.float32),
    mesh=_mesh,
    scratch_types=[
        pltpu.VMEM((ROWS_PER_TILE,), jnp.int32),
        pltpu.VMEM((NUM_TILES, EMB_DIM), jnp.float32),
        pltpu.VMEM((ROWS_PER_TILE, EMB_DIM), jnp.float32),
        pltpu.VMEM((EMB_DIM,), jnp.float32),
        pltpu.VMEM_SHARED((NUM_TILES, EMB_DIM), jnp.float32),
        pltpu.VMEM((1, EMB_DIM), jnp.float32),
        pltpu.SemaphoreType.DMA,
    ],
)


def kernel(input_sequence, emb_table, f):
    idx = input_sequence.astype(jnp.int32)
    idx_pad = jnp.zeros((NUM_TILES * ROWS_PER_TILE,), jnp.int32)
    idx_pad = idx_pad.at[:SEQ_LEN].set(idx)
    f_pad = jnp.zeros((NUM_TILES * ROWS_PER_TILE, EMB_DIM), jnp.float32)
    f_pad = f_pad.at[:SEQ_LEN, :].set(f)
    return _sc_call(idx_pad, emb_table, f_pad)


# speedup vs baseline: 1.1036x; 1.1036x over previous
import jax
import jax.numpy as jnp
from jax import lax
from jax.experimental import pallas as pl
from jax.experimental.pallas import tpu as pltpu
from jax.experimental.pallas import tpu_sc as plsc

SEQ_LEN = 200
EMB_DIM = 128
LANES = 16
NUM_TILES = 16
FULL_ROWS = 16
TAIL_TILE = 12
TAIL_ROWS = SEQ_LEN - TAIL_TILE * FULL_ROWS
ACTIVE_TILES = TAIL_TILE + 1
CHUNKS = EMB_DIM // LANES

_mesh = plsc.VectorSubcoreMesh(
    core_axis_name="c", subcore_axis_name="s", num_cores=1
)


def _sc_body(idx_hbm, table_hbm, f_hbm, out_hbm,
             idx_v, rows_v, f_v, idx8_v, rows8_v, f8_v,
             part_v, shared, res_v, sem_i, sem_f, sem_g):

    sid = lax.axis_index("s")

    @pl.when(sid < TAIL_TILE)
    def _():
        base = sid * FULL_ROWS
        cp_i = pltpu.async_copy(idx_hbm.at[pl.ds(base, FULL_ROWS)], idx_v, sem_i)
        cp_f = pltpu.async_copy(f_hbm.at[pl.ds(base, FULL_ROWS)], f_v, sem_f)
        cp_i.wait()
        cp_g = pltpu.async_copy(table_hbm.at[idx_v], rows_v, sem_g)
        cp_f.wait()
        cp_g.wait()
        for c in range(CHUNKS):
            col = pl.ds(c * LANES, LANES)
            acc = rows_v[0, col] * f_v[0, col]
            for r in range(1, FULL_ROWS):
                acc = acc + rows_v[r, col] * f_v[r, col]
            part_v[col] = acc
        pltpu.sync_copy(part_v, shared.at[sid])

    @pl.when(sid == TAIL_TILE)
    def _():
        base = TAIL_TILE * FULL_ROWS
        cp_i = pltpu.async_copy(idx_hbm.at[pl.ds(base, TAIL_ROWS)], idx8_v, sem_i)
        cp_f = pltpu.async_copy(f_hbm.at[pl.ds(base, TAIL_ROWS)], f8_v, sem_f)
        cp_i.wait()
        cp_g = pltpu.async_copy(table_hbm.at[idx8_v], rows8_v, sem_g)
        cp_f.wait()
        cp_g.wait()
        for c in range(CHUNKS):
            col = pl.ds(c * LANES, LANES)
            acc = rows8_v[0, col] * f8_v[0, col]
            for r in range(1, TAIL_ROWS):
                acc = acc + rows8_v[r, col] * f8_v[r, col]
            part_v[col] = acc
        pltpu.sync_copy(part_v, shared.at[TAIL_TILE])

    plsc.subcore_barrier()

    @pl.when(sid == 0)
    def _():
        pltpu.sync_copy(shared.at[pl.ds(0, ACTIVE_TILES)], rows_v.at[pl.ds(0, ACTIVE_TILES)])
        for c in range(CHUNKS):
            col = pl.ds(c * LANES, LANES)
            s = rows_v[0, col]
            for t in range(1, ACTIVE_TILES):
                s = s + rows_v[t, col]
            res_v[0, col] = s
        pltpu.sync_copy(res_v, out_hbm)


_sc_call = pl.kernel(
    _sc_body,
    out_type=jax.ShapeDtypeStruct((1, EMB_DIM), jnp.float32),
    mesh=_mesh,
    scratch_types=[
        pltpu.VMEM((FULL_ROWS,), jnp.int32),
        pltpu.VMEM((FULL_ROWS, EMB_DIM), jnp.float32),
        pltpu.VMEM((FULL_ROWS, EMB_DIM), jnp.float32),
        pltpu.VMEM((TAIL_ROWS,), jnp.int32),
        pltpu.VMEM((TAIL_ROWS, EMB_DIM), jnp.float32),
        pltpu.VMEM((TAIL_ROWS, EMB_DIM), jnp.float32),
        pltpu.VMEM((EMB_DIM,), jnp.float32),
        pltpu.VMEM_SHARED((NUM_TILES, EMB_DIM), jnp.float32),
        pltpu.VMEM((1, EMB_DIM), jnp.float32),
        pltpu.SemaphoreType.DMA,
        pltpu.SemaphoreType.DMA,
        pltpu.SemaphoreType.DMA,
    ],
)


def kernel(input_sequence, emb_table, f):
    return _sc_call(input_sequence.astype(jnp.int32), emb_table, f)


# speedup vs baseline: 1.1303x; 1.0242x over previous
import jax
import jax.numpy as jnp
from jax import lax
from jax.experimental import pallas as pl
from jax.experimental.pallas import tpu as pltpu
from jax.experimental.pallas import tpu_sc as plsc

SEQ_LEN = 200
EMB_DIM = 128
LANES = 16
ROWS = 16
HALF = ROWS // 2
TAIL_TILE = 12
TAIL_BASE = SEQ_LEN - ROWS
ACTIVE_TILES = 13
CHUNKS = EMB_DIM // LANES

_mesh = plsc.VectorSubcoreMesh(
    core_axis_name="c", subcore_axis_name="s", num_cores=1
)


def _sc_body(idx_hbm, table_hbm, f_hbm, zi_hbm, out_hbm,
             idx_v, rows_v, f_v, part_v, zi_v, zero_v, shared,
             sem_i, sem_f, sem_g, sem_z):
    sid = lax.axis_index("s")
    base = jnp.where(sid < TAIL_TILE, sid * ROWS,
                     jnp.where(sid == TAIL_TILE, TAIL_BASE, 0))

    cp_i = pltpu.async_copy(idx_hbm.at[pl.ds(base, ROWS)], idx_v, sem_i)
    cp_f = pltpu.async_copy(f_hbm.at[pl.ds(base, ROWS)], f_v, sem_f)
    cp_z = pltpu.async_copy(zi_hbm, zi_v, sem_z)

    @pl.when(sid == 15)
    def _():
        for c in range(CHUNKS):
            zero_v[0, pl.ds(c * LANES, LANES)] = jnp.zeros((LANES,), jnp.float32)
        pltpu.sync_copy(zero_v, shared)

    plsc.subcore_barrier()

    cp_i.wait()
    cp_g = pltpu.async_copy(table_hbm.at[idx_v], rows_v, sem_g)
    cp_f.wait()
    cp_g.wait()

    wlo = jnp.where(sid == TAIL_TILE, 0.0, 1.0)
    for c in range(CHUNKS):
        col = pl.ds(c * LANES, LANES)
        alo = rows_v[0, col] * f_v[0, col]
        for r in range(1, HALF):
            alo = alo + rows_v[r, col] * f_v[r, col]
        ahi = rows_v[HALF, col] * f_v[HALF, col]
        for r in range(HALF + 1, ROWS):
            ahi = ahi + rows_v[r, col] * f_v[r, col]
        part_v[0, col] = ahi + wlo * alo

    cp_z.wait()

    @pl.when(sid < ACTIVE_TILES)
    def _():
        pltpu.sync_copy(part_v, shared.at[zi_v], add=True)

    plsc.subcore_barrier()

    @pl.when(sid == 0)
    def _():
        pltpu.sync_copy(shared, out_hbm)


_sc_call = pl.kernel(
    _sc_body,
    out_type=jax.ShapeDtypeStruct((1, EMB_DIM), jnp.float32),
    mesh=_mesh,
    scratch_types=[
        pltpu.VMEM((ROWS,), jnp.int32),
        pltpu.VMEM((ROWS, EMB_DIM), jnp.float32),
        pltpu.VMEM((ROWS, EMB_DIM), jnp.float32),
        pltpu.VMEM((1, EMB_DIM), jnp.float32),
        pltpu.VMEM((1,), jnp.int32),
        pltpu.VMEM((1, EMB_DIM), jnp.float32),
        pltpu.VMEM_SHARED((1, EMB_DIM), jnp.float32),
        pltpu.SemaphoreType.DMA,
        pltpu.SemaphoreType.DMA,
        pltpu.SemaphoreType.DMA,
        pltpu.SemaphoreType.DMA,
    ],
)


def kernel(input_sequence, emb_table, f):
    zero_idx = jnp.zeros((1,), jnp.int32)
    return _sc_call(input_sequence.astype(jnp.int32), emb_table, f, zero_idx)
